# Initial kernel scaffold; baseline (speedup 1.0000x reference)
import jax
import jax.numpy as jnp
from jax.experimental import pallas as pl


def kernel(hfeats, cfeats, efeats, edge_index, W_ne, b_ne, W_ee, b_ee, W_ce, b_ce, W_n1, b_n1, W_n2, b_n2, W_e1, b_e1, W_c1, b_c1):
    raise NotImplementedError("write your pallas kernel here")


# speedup vs baseline: 4.3114x; 4.3114x over previous
import functools
import jax
import jax.numpy as jnp
from jax import lax
from jax.experimental import pallas as pl
from jax.experimental.pallas import tpu as pltpu
from jax.experimental.pallas import tpu_sc as plsc


def _celu(x):
    return jnp.where(x > 0, x, jnp.exp(x) - 1.0)


def _node_prep_body(hf_ref, cfp_ref, wne_ref, bne_ref, wcep_ref, bce_ref,
                    h_ref, c_ref):
    h_ref[...] = _celu(
        jnp.dot(hf_ref[...], wne_ref[...], preferred_element_type=jnp.float32)
        + bne_ref[...])
    c_ref[...] = _celu(
        jnp.dot(cfp_ref[...], wcep_ref[...], preferred_element_type=jnp.float32)
        + bce_ref[...])


def _node_prep(hfeats, cf_pad, W_ne, b_ne, W_ce_pad, b_ce, n_blk):
    n = hfeats.shape[0]
    dn = hfeats.shape[1]
    h_dim = W_ne.shape[1]
    dcp = cf_pad.shape[1]
    grid = n // n_blk
    return pl.pallas_call(
        _node_prep_body,
        grid=(grid,),
        in_specs=[
            pl.BlockSpec((n_blk, dn), lambda i: (i, 0)),
            pl.BlockSpec((n_blk, dcp), lambda i: (i, 0)),
            pl.BlockSpec((dn, h_dim), lambda i: (0, 0)),
            pl.BlockSpec((1, h_dim), lambda i: (0, 0)),
            pl.BlockSpec((dcp, h_dim), lambda i: (0, 0)),
            pl.BlockSpec((1, h_dim), lambda i: (0, 0)),
        ],
        out_specs=[
            pl.BlockSpec((n_blk, h_dim), lambda i: (i, 0)),
            pl.BlockSpec((n_blk, h_dim), lambda i: (i, 0)),
        ],
        out_shape=[
            jax.ShapeDtypeStruct((n, h_dim), jnp.float32),
            jax.ShapeDtypeStruct((n, h_dim), jnp.float32),
        ],
    )(hfeats, cf_pad, W_ne, b_ne.reshape(1, -1), W_ce_pad, b_ce.reshape(1, -1))


def _edge_prep_body(ef_ref, wee_ref, bee_ref, we1_ref, be1_ref,
                    e_ref, efo_ref):
    ef = ef_ref[...]
    e = _celu(
        jnp.dot(ef, wee_ref[...], preferred_element_type=jnp.float32)
        + bee_ref[...])
    e_ref[...] = e
    efo_ref[...] = ef + _celu(
        jnp.dot(e, we1_ref[...], preferred_element_type=jnp.float32)
        + be1_ref[...])


def _edge_prep(efeats, W_ee, b_ee, W_e1, b_e1, e_blk):
    e_edges = efeats.shape[0]
    de = efeats.shape[1]
    h_dim = W_ee.shape[1]
    grid = e_edges // e_blk
    return pl.pallas_call(
        _edge_prep_body,
        grid=(grid,),
        in_specs=[
            pl.BlockSpec((e_blk, de), lambda i: (i, 0)),
            pl.BlockSpec((de, h_dim), lambda i: (0, 0)),
            pl.BlockSpec((1, h_dim), lambda i: (0, 0)),
            pl.BlockSpec((h_dim, de), lambda i: (0, 0)),
            pl.BlockSpec((1, de), lambda i: (0, 0)),
        ],
        out_specs=[
            pl.BlockSpec((e_blk, h_dim), lambda i: (i, 0)),
            pl.BlockSpec((e_blk, de), lambda i: (i, 0)),
        ],
        out_shape=[
            jax.ShapeDtypeStruct((e_edges, h_dim), jnp.float32),
            jax.ShapeDtypeStruct((e_edges, de), jnp.float32),
        ],
    )(efeats, W_ee, b_ee.reshape(1, -1), W_e1, b_e1.reshape(1, -1))


def _sc_edge_kernel(src, dst, e, h, c, n_pad):
    e_edges, h_dim = e.shape
    nc, ns, nl = 2, 16, 16
    nw = nc * ns
    epw = e_edges // nw
    B = 40
    nb = epw // B
    rows_per_tile = n_pad // ns
    wchunk = 64
    nq = rows_per_tile // wchunk
    nvec = h_dim // nl

    mesh = plsc.VectorSubcoreMesh(core_axis_name="c", subcore_axis_name="s")

    @functools.partial(
        pl.kernel,
        out_type=jax.ShapeDtypeStruct((nc, n_pad, h_dim), jnp.float32),
        mesh=mesh,
        scratch_types=[
            pltpu.VMEM((B,), jnp.int32),
            pltpu.VMEM((B,), jnp.int32),
            pltpu.VMEM((B, h_dim), jnp.float32),
            pltpu.VMEM((B, h_dim), jnp.float32),
            pltpu.VMEM((B, h_dim), jnp.float32),
            pltpu.VMEM((B, h_dim), jnp.float32),
            pltpu.VMEM((B, h_dim), jnp.float32),
            pltpu.VMEM((wchunk, h_dim), jnp.float32),
            pltpu.VMEM_SHARED((n_pad, h_dim), jnp.float32),
            pltpu.SemaphoreType.DMA,
            pltpu.SemaphoreType.DMA,
            pltpu.SemaphoreType.DMA,
            pltpu.SemaphoreType.DMA,
        ],
    )
    def body(src_hbm, dst_hbm, e_hbm, h_hbm, c_hbm, kout_hbm,
             srcv, dstv, hv, csv, cdv, ev, xv, zv, ksh,
             sem0, sem1, sem2, sem3):
        cid = lax.axis_index("c")
        sid = lax.axis_index("s")
        wid = sid * nc + cid
        base = wid * epw
        row0 = sid * rows_per_tile

        def zrow(i, carry):
            for j in range(nvec):
                zv[i, pl.ds(j * nl, nl)] = jnp.zeros((nl,), jnp.float32)
            return carry
        lax.fori_loop(0, wchunk, zrow, 0)
        for q in range(nq):
            pltpu.sync_copy(zv, ksh.at[pl.ds(row0 + q * wchunk, wchunk)])
        plsc.subcore_barrier()

        def step(t, carry):
            off = base + t * B
            cp_e = pltpu.async_copy(e_hbm.at[pl.ds(off, B)], ev, sem3)
            pltpu.sync_copy(src_hbm.at[pl.ds(off, B)], srcv)
            pltpu.sync_copy(dst_hbm.at[pl.ds(off, B)], dstv)
            cp_h = pltpu.async_copy(h_hbm.at[srcv], hv, sem0)
            cp_cs = pltpu.async_copy(c_hbm.at[srcv], csv, sem1)
            cp_cd = pltpu.async_copy(c_hbm.at[dstv], cdv, sem2)
            cp_e.wait()
            cp_h.wait()
            cp_cs.wait()
            cp_cd.wait()

            def crow(i, carry2):
                for j in range(nvec):
                    sl = pl.ds(j * nl, nl)
                    diff = jnp.abs(cdv[i, sl] - csv[i, sl])
                    xv[i, sl] = diff * (hv[i, sl] * ev[i, sl])
                return carry2
            lax.fori_loop(0, B, crow, 0)

            pltpu.sync_copy(xv, ksh.at[dstv], add=True)
            return carry
        lax.fori_loop(0, nb, step, 0)

        plsc.subcore_barrier()
        for q in range(nq):
            r = row0 + q * wchunk
            pltpu.sync_copy(ksh.at[pl.ds(r, wchunk)], zv)
            pltpu.sync_copy(zv, kout_hbm.at[cid, pl.ds(r, wchunk)])

    return body(src, dst, e, h, c)


def _node_post_body(k0_ref, k1_ref, hf_ref, cfp_ref, c_ref,
                    wn1_ref, bn1_ref, wn2_ref, bn2_ref, wc1p_ref, bc1p_ref,
                    hfo_ref, cfo_ref):
    k = k0_ref[...] + k1_ref[...]
    t = _celu(
        jnp.dot(k, wn1_ref[...], preferred_element_type=jnp.float32)
        + bn1_ref[...])
    hfo_ref[...] = hf_ref[...] + _celu(
        jnp.dot(t, wn2_ref[...], preferred_element_type=jnp.float32)
        + bn2_ref[...])
    cfo_ref[...] = cfp_ref[...] + _celu(
        jnp.dot(c_ref[...], wc1p_ref[...], preferred_element_type=jnp.float32)
        + bc1p_ref[...])


def _node_post(k0, k1, hfeats, cf_pad, c, W_n1, b_n1, W_n2, b_n2,
               W_c1_pad, b_c1_pad, n_blk):
    n, h_dim = k0.shape
    dn = hfeats.shape[1]
    dcp = cf_pad.shape[1]
    grid = n // n_blk
    return pl.pallas_call(
        _node_post_body,
        grid=(grid,),
        in_specs=[
            pl.BlockSpec((n_blk, h_dim), lambda i: (i, 0)),
            pl.BlockSpec((n_blk, h_dim), lambda i: (i, 0)),
            pl.BlockSpec((n_blk, dn), lambda i: (i, 0)),
            pl.BlockSpec((n_blk, dcp), lambda i: (i, 0)),
            pl.BlockSpec((n_blk, h_dim), lambda i: (i, 0)),
            pl.BlockSpec((h_dim, h_dim), lambda i: (0, 0)),
            pl.BlockSpec((1, h_dim), lambda i: (0, 0)),
            pl.BlockSpec((h_dim, dn), lambda i: (0, 0)),
            pl.BlockSpec((1, dn), lambda i: (0, 0)),
            pl.BlockSpec((h_dim, dcp), lambda i: (0, 0)),
            pl.BlockSpec((1, dcp), lambda i: (0, 0)),
        ],
        out_specs=[
            pl.BlockSpec((n_blk, dn), lambda i: (i, 0)),
            pl.BlockSpec((n_blk, dcp), lambda i: (i, 0)),
        ],
        out_shape=[
            jax.ShapeDtypeStruct((n, dn), jnp.float32),
            jax.ShapeDtypeStruct((n, dcp), jnp.float32),
        ],
    )(k0, k1, hfeats, cf_pad, c, W_n1, b_n1.reshape(1, -1), W_n2,
      b_n2.reshape(1, -1), W_c1_pad, b_c1_pad.reshape(1, -1))


def kernel(hfeats, cfeats, efeats, edge_index, W_ne, b_ne, W_ee, b_ee, W_ce,
           b_ce, W_n1, b_n1, W_n2, b_n2, W_e1, b_e1, W_c1, b_c1):
    n = hfeats.shape[0]
    dc = cfeats.shape[1]
    dcp = 8

    src = edge_index[0].astype(jnp.int32)
    dst = edge_index[1].astype(jnp.int32)

    cf_pad = jnp.pad(cfeats, ((0, 0), (0, dcp - dc)))
    W_ce_pad = jnp.pad(W_ce, ((0, dcp - dc), (0, 0)))
    W_c1_pad = jnp.pad(W_c1, ((0, 0), (0, dcp - dc)))
    b_c1_pad = jnp.pad(b_c1, (0, dcp - dc))

    h, c = _node_prep(hfeats, cf_pad, W_ne, b_ne, W_ce_pad, b_ce, n_blk=1000)
    e, efeats_out = _edge_prep(efeats, W_ee, b_ee, W_e1, b_e1, e_blk=2000)

    n_pad = 10240
    k_parts = _sc_edge_kernel(src, dst, e, h, c, n_pad)

    hfeats_out, cf_out_pad = _node_post(
        k_parts[0, :n], k_parts[1, :n], hfeats, cf_pad, c,
        W_n1, b_n1, W_n2, b_n2, W_c1_pad, b_c1_pad, n_blk=1000)
    cfeats_out = cf_out_pad[:, :dc]
    return (hfeats_out, cfeats_out, efeats_out)
